# baseline (speedup 1.0000x reference)
import functools

import jax
import jax.numpy as jnp
from jax import lax
from jax.experimental import pallas as pl
from jax.experimental.pallas import tpu as pltpu
from jax.experimental.pallas import tpu_sc as plsc

B = 16384
HIST = 50
D = 64
NC = 2
NS = 16
NW = NC * NS
BPW = B // NW
LROWS = B // 128
LPW = BPW // 128


VOCAB = 1000001
TCH = 384
TFULL = VOCAB // TCH
TTAIL = VOCAB - TFULL * TCH
TEXTRA = TFULL - (TFULL // NW) * NW


TCB = 2048
TNB = 2 * (-(-VOCAB // (2 * TCB)))
VPAD = TNB * TCB


def _tc_transpose_body(xa_ref, xb_ref, o_ref):
    o_ref[...] = jnp.concatenate(
        [jnp.transpose(xa_ref[...]), jnp.transpose(xb_ref[...])], axis=1)


def _tc_transpose(table_t):
    nblk = VPAD // 2 // TCB
    return pl.pallas_call(
        _tc_transpose_body,
        grid=(nblk,),
        in_specs=[
            pl.BlockSpec((D, TCB), lambda i: (0, i)),
            pl.BlockSpec(
                (D, TCB),
                lambda i: (0, jnp.minimum(i + VPAD // 2 // TCB,
                                          (VOCAB - 1) // TCB)),
            ),
        ],
        out_specs=pl.BlockSpec((TCB, 2 * D), lambda i: (i, 0)),
        out_shape=jax.ShapeDtypeStruct((VPAD // 2, 2 * D), jnp.float32),
    )(table_t, table_t)


def _sc_transpose(table_t, tail_flat):
    mesh = plsc.VectorSubcoreMesh(core_axis_name="c", subcore_axis_name="s")

    @functools.partial(
        pl.kernel,
        out_type=jax.ShapeDtypeStruct((VOCAB * D,), jnp.float32),
        mesh=mesh,
        compiler_params=pltpu.CompilerParams(needs_layout_passes=False),
        scratch_types=[
            pltpu.VMEM((2, D, TCH + 1), jnp.float32),
            pltpu.VMEM((2, TCH * D), jnp.float32),
            pltpu.VMEM((TTAIL * D,), jnp.float32),
            pltpu.SemaphoreType.DMA,
            pltpu.SemaphoreType.DMA,
            pltpu.SemaphoreType.DMA,
            pltpu.SemaphoreType.DMA,
        ],
    )
    def k(tt_hbm, tail_hbm, out_hbm, in_v, out_v, tail_v, si0, si1, so0, so1):
        wid = lax.axis_index("s") * NC + lax.axis_index("c")
        n = (TFULL // NW) + jnp.where(wid < TEXTRA, 1, 0).astype(jnp.int32)

        iotas = [lax.iota(jnp.int32, 16) + 16 * kk for kk in range(D // 16)]

        def in_band(j, buf, t, sem):
            return pltpu.make_async_copy(
                tt_hbm.at[pl.ds(8 * t, 8), pl.ds((wid + NW * j) * TCH, TCH)],
                in_v.at[buf, pl.ds(8 * t, 8), pl.ds(0, TCH)], sem)

        def in_dma_start(j, buf, sem):
            for t in range(D // 8):
                in_band(j, buf, t, sem).start()

        def in_dma_wait(j, buf, sem):
            for t in range(D // 8):
                in_band(j, buf, t, sem).wait()

        def out_dma(j, buf, sem):
            return pltpu.make_async_copy(
                out_v.at[buf],
                out_hbm.at[pl.ds((wid + NW * j) * TCH * D, TCH * D)], sem)

        def transpose(buf):
            def row(i, carry):
                for u in range(2):
                    ii = 2 * i + u
                    cidx = jnp.full((16,), 0, jnp.int32) + ii
                    for kk in range(D // 16):
                        v = plsc.load_gather(in_v.at[buf], [iotas[kk], cidx])
                        out_v[buf, pl.ds(ii * D + 16 * kk, 16)] = v
                return carry
            lax.fori_loop(0, TCH // 2, row, 0)

        in_dma_start(0, 0, si0)

        def body(m, carry):
            j0 = 2 * m
            j1 = j0 + 1

            @pl.when(j1 < n)
            def _():
                in_dma_start(j1, 1, si1)

            @pl.when(j0 < n)
            def _():
                in_dma_wait(j0, 0, si0)

                @pl.when(m > 0)
                def _():
                    out_dma(j0 - 2, 0, so0).wait()

                transpose(0)
                out_dma(j0, 0, so0).start()

            @pl.when(j0 + 2 < n)
            def _():
                in_dma_start(j0 + 2, 0, si0)

            @pl.when(j1 < n)
            def _():
                in_dma_wait(j1, 1, si1)

                @pl.when(m > 0)
                def _():
                    out_dma(j1 - 2, 1, so1).wait()

                transpose(1)
                out_dma(j1, 1, so1).start()

            return carry

        lax.fori_loop(0, (TFULL // NW + 1 + 1) // 2, body, 0)

        out_dma(0, 0, so0).wait()
        out_dma(1, 1, so1).wait()

        @pl.when(wid == TEXTRA)
        def _():
            pltpu.sync_copy(tail_hbm, tail_v)
            pltpu.sync_copy(tail_v,
                            out_hbm.at[pl.ds(TFULL * TCH * D, TTAIL * D)])

    return k(table_t, tail_flat)


def _sc_gather_pool(symp, label2d, symp_table, dise_table):
    mesh = plsc.VectorSubcoreMesh(core_axis_name="c", subcore_axis_name="s")

    @functools.partial(
        pl.kernel,
        out_type=(
            jax.ShapeDtypeStruct((B, D), jnp.float32),
            jax.ShapeDtypeStruct((B, D), jnp.float32),
        ),
        mesh=mesh,
        compiler_params=pltpu.CompilerParams(use_tc_tiling_on_sc=False),
        scratch_types=[
            pltpu.VMEM((BPW, HIST), jnp.int32),
            pltpu.VMEM((4, HIST, D), jnp.float32),
            pltpu.VMEM((BPW, D), jnp.float32),
            pltpu.VMEM((LPW, 128), jnp.int32),
            pltpu.VMEM((BPW, D), jnp.float32),
            pltpu.SemaphoreType.DMA,
            pltpu.SemaphoreType.DMA,
            pltpu.SemaphoreType.DMA,
            pltpu.SemaphoreType.DMA,
            pltpu.SemaphoreType.DMA,
        ],
    )
    def k(symp_hbm, label_hbm, stab_hbm, dtab_hbm, out_u_hbm, out_d_hbm,
          idx_v, rows_v, outu_v, lidx_v, drows_v, sem0, sem1, sem2, sem3, semd):
        wid = lax.axis_index("s") * NC + lax.axis_index("c")
        base = wid * BPW

        pltpu.sync_copy(symp_hbm.at[pl.ds(base, BPW)], idx_v)
        pltpu.sync_copy(label_hbm.at[pl.ds(wid * LPW, LPW)], lidx_v)

        for j in range(LPW):
            pltpu.async_copy(
                dtab_hbm.at[lidx_v.at[j]], drows_v.at[pl.ds(j * 128, 128)], semd)

        def accum(buf, b):
            for d in range(D // 16):
                acc = rows_v[buf, 0, pl.ds(d * 16, 16)]
                for r in range(1, HIST):
                    acc = acc + rows_v[buf, r, pl.ds(d * 16, 16)]
                outu_v[b, pl.ds(d * 16, 16)] = acc

        sems = [sem0, sem1, sem2, sem3]

        for q in range(3):
            pltpu.async_copy(stab_hbm.at[idx_v.at[q]], rows_v.at[q], sems[q])

        def body(i, carry):
            b0 = 4 * i
            pltpu.async_copy(
                stab_hbm.at[idx_v.at[b0 + 3]], rows_v.at[3], sems[3])
            for q in range(4):
                b = b0 + q
                pltpu.make_async_copy(
                    stab_hbm.at[idx_v.at[b]], rows_v.at[q], sems[q]).wait()
                accum(q, b)

                if q < 3:
                    @pl.when(b + 4 < BPW)
                    def _():
                        pltpu.async_copy(
                            stab_hbm.at[idx_v.at[b + 4]], rows_v.at[q],
                            sems[q])
            return carry

        lax.fori_loop(0, BPW // 4, body, 0)

        pltpu.sync_copy(outu_v, out_u_hbm.at[pl.ds(base, BPW)])
        for j in range(LPW):
            pltpu.make_async_copy(
                dtab_hbm.at[lidx_v.at[j]], drows_v.at[pl.ds(j * 128, 128)],
                semd).wait()
        pltpu.sync_copy(drows_v, out_d_hbm.at[pl.ds(base, BPW)])

    return k(symp, label2d, symp_table, dise_table)


def _mlp_body(symp_ref, su_ref, sd_ref, w1u_ref, w1d_ref, b1_ref, w2_ref,
              b2_ref, o_ref):
    cnt = jnp.sum((symp_ref[...] != 0).astype(jnp.float32), axis=1,
                  keepdims=True)
    w = 1.0 / (cnt + 1e-8)
    w = jnp.where(w >= 1e8, 0.0, w)
    u = jnp.maximum(su_ref[...] * w, 0.0)
    d = jnp.maximum(sd_ref[...], 0.0)
    h = (jnp.dot(u, w1u_ref[...], preferred_element_type=jnp.float32)
         + jnp.dot(d, w1d_ref[...], preferred_element_type=jnp.float32)
         + b1_ref[...])
    h = jnp.maximum(h, 0.0)
    o_ref[...] = jnp.sum(h * w2_ref[...], axis=1, keepdims=True) + b2_ref[...]


def _mlp(symp, sum_u, sum_d, W1, b1, W2, b2):
    BLK = 2048
    return pl.pallas_call(
        _mlp_body,
        grid=(B // BLK,),
        in_specs=[
            pl.BlockSpec((BLK, HIST), lambda i: (i, 0)),
            pl.BlockSpec((BLK, D), lambda i: (i, 0)),
            pl.BlockSpec((BLK, D), lambda i: (i, 0)),
            pl.BlockSpec((D, D), lambda i: (0, 0)),
            pl.BlockSpec((D, D), lambda i: (0, 0)),
            pl.BlockSpec((1, D), lambda i: (0, 0)),
            pl.BlockSpec((1, D), lambda i: (0, 0)),
            pl.BlockSpec((1, 1), lambda i: (0, 0)),
        ],
        out_specs=pl.BlockSpec((BLK, 1), lambda i: (i, 0)),
        out_shape=jax.ShapeDtypeStruct((B, 1), jnp.float32),
    )(symp, sum_u, sum_d, W1[:D], W1[D:], b1.reshape(1, D),
      W2.reshape(1, D), b2.reshape(1, 1))


def kernel(symp, label, symp_table, dise_table, W1, b1, W2, b2):
    symp_i = symp.astype(jnp.int32)
    label2d = label.astype(jnp.int32).reshape(LROWS, 128)
    h = VPAD // 2
    symp_r = jnp.where(symp_i < h, 2 * symp_i, 2 * (symp_i - h) + 1)
    table_lin = _tc_transpose(symp_table.T).reshape(VPAD, D)
    sum_u, sum_d = _sc_gather_pool(symp_r, label2d, table_lin, dise_table)
    return _mlp(symp_i, sum_u, sum_d, W1, b1, W2, b2)


# speedup vs baseline: 1.0347x; 1.0347x over previous
import functools

import jax
import jax.numpy as jnp
from jax import lax
from jax.experimental import pallas as pl
from jax.experimental.pallas import tpu as pltpu
from jax.experimental.pallas import tpu_sc as plsc

B = 16384
HIST = 50
D = 64
NC = 2
NS = 16
NW = NC * NS
BPW = B // NW
LROWS = B // 128
LPW = BPW // 128


VOCAB = 1000001
TCB = 2048
TNB = 2 * (-(-VOCAB // (2 * TCB)))
VPAD = TNB * TCB


def _tc_transpose_body(xa_ref, xb_ref, o_ref):
    o_ref[...] = jnp.concatenate(
        [jnp.transpose(xa_ref[...]), jnp.transpose(xb_ref[...])], axis=1)


def _tc_transpose(table_t):
    nblk = VPAD // 2 // TCB
    return pl.pallas_call(
        _tc_transpose_body,
        grid=(nblk,),
        in_specs=[
            pl.BlockSpec((D, TCB), lambda i: (0, i)),
            pl.BlockSpec(
                (D, TCB),
                lambda i: (0, jnp.minimum(i + VPAD // 2 // TCB,
                                          (VOCAB - 1) // TCB)),
            ),
        ],
        out_specs=pl.BlockSpec((TCB, 2 * D), lambda i: (i, 0)),
        out_shape=jax.ShapeDtypeStruct((VPAD // 2, 2 * D), jnp.float32),
    )(table_t, table_t)


def _sc_gather_pool(symp, label2d, symp_table, dise_table):
    mesh = plsc.VectorSubcoreMesh(core_axis_name="c", subcore_axis_name="s")

    @functools.partial(
        pl.kernel,
        out_type=(
            jax.ShapeDtypeStruct((B, D), jnp.float32),
            jax.ShapeDtypeStruct((B, D), jnp.float32),
        ),
        mesh=mesh,
        compiler_params=pltpu.CompilerParams(use_tc_tiling_on_sc=False),
        scratch_types=[
            pltpu.VMEM((BPW, HIST), jnp.int32),
            pltpu.VMEM((2, HIST, D), jnp.float32),
            pltpu.VMEM((BPW, D), jnp.float32),
            pltpu.VMEM((LPW, 128), jnp.int32),
            pltpu.VMEM((BPW, D), jnp.float32),
            pltpu.SemaphoreType.DMA,
            pltpu.SemaphoreType.DMA,
            pltpu.SemaphoreType.DMA,
        ],
    )
    def k(symp_hbm, label_hbm, stab_hbm, dtab_hbm, out_u_hbm, out_d_hbm,
          idx_v, rows_v, outu_v, lidx_v, drows_v, sem0, sem1, semd):
        wid = lax.axis_index("s") * NC + lax.axis_index("c")
        base = wid * BPW

        pltpu.sync_copy(symp_hbm.at[pl.ds(base, BPW)], idx_v)
        pltpu.sync_copy(label_hbm.at[pl.ds(wid * LPW, LPW)], lidx_v)

        for j in range(LPW):
            pltpu.async_copy(
                dtab_hbm.at[lidx_v.at[j]], drows_v.at[pl.ds(j * 128, 128)], semd)

        def accum(buf, b):
            for d in range(D // 16):
                acc = rows_v[buf, 0, pl.ds(d * 16, 16)]
                for r in range(1, HIST):
                    acc = acc + rows_v[buf, r, pl.ds(d * 16, 16)]
                outu_v[b, pl.ds(d * 16, 16)] = acc

        pltpu.async_copy(stab_hbm.at[idx_v.at[0]], rows_v.at[0], sem0)

        def body(i, carry):
            b0 = 2 * i
            b1 = b0 + 1
            pltpu.async_copy(stab_hbm.at[idx_v.at[b1]], rows_v.at[1], sem1)
            pltpu.make_async_copy(
                stab_hbm.at[idx_v.at[b0]], rows_v.at[0], sem0).wait()
            accum(0, b0)

            @pl.when(i + 1 < BPW // 2)
            def _():
                pltpu.async_copy(
                    stab_hbm.at[idx_v.at[b0 + 2]], rows_v.at[0], sem0)

            pltpu.make_async_copy(
                stab_hbm.at[idx_v.at[b1]], rows_v.at[1], sem1).wait()
            accum(1, b1)
            return carry

        lax.fori_loop(0, BPW // 2, body, 0)

        pltpu.sync_copy(outu_v, out_u_hbm.at[pl.ds(base, BPW)])
        for j in range(LPW):
            pltpu.make_async_copy(
                dtab_hbm.at[lidx_v.at[j]], drows_v.at[pl.ds(j * 128, 128)],
                semd).wait()
        pltpu.sync_copy(drows_v, out_d_hbm.at[pl.ds(base, BPW)])

    return k(symp, label2d, symp_table, dise_table)


def _mlp_body(symp_ref, su_ref, sd_ref, w1u_ref, w1d_ref, b1_ref, w2_ref,
              b2_ref, o_ref):
    cnt = jnp.sum((symp_ref[...] != 0).astype(jnp.float32), axis=1,
                  keepdims=True)
    w = 1.0 / (cnt + 1e-8)
    w = jnp.where(w >= 1e8, 0.0, w)
    u = jnp.maximum(su_ref[...] * w, 0.0)
    d = jnp.maximum(sd_ref[...], 0.0)
    h = (jnp.dot(u, w1u_ref[...], preferred_element_type=jnp.float32)
         + jnp.dot(d, w1d_ref[...], preferred_element_type=jnp.float32)
         + b1_ref[...])
    h = jnp.maximum(h, 0.0)
    o_ref[...] = jnp.sum(h * w2_ref[...], axis=1, keepdims=True) + b2_ref[...]


def _mlp(symp, sum_u, sum_d, W1, b1, W2, b2):
    BLK = 2048
    return pl.pallas_call(
        _mlp_body,
        grid=(B // BLK,),
        in_specs=[
            pl.BlockSpec((BLK, HIST), lambda i: (i, 0)),
            pl.BlockSpec((BLK, D), lambda i: (i, 0)),
            pl.BlockSpec((BLK, D), lambda i: (i, 0)),
            pl.BlockSpec((D, D), lambda i: (0, 0)),
            pl.BlockSpec((D, D), lambda i: (0, 0)),
            pl.BlockSpec((1, D), lambda i: (0, 0)),
            pl.BlockSpec((1, D), lambda i: (0, 0)),
            pl.BlockSpec((1, 1), lambda i: (0, 0)),
        ],
        out_specs=pl.BlockSpec((BLK, 1), lambda i: (i, 0)),
        out_shape=jax.ShapeDtypeStruct((B, 1), jnp.float32),
    )(symp, sum_u, sum_d, W1[:D], W1[D:], b1.reshape(1, D),
      W2.reshape(1, D), b2.reshape(1, 1))


def kernel(symp, label, symp_table, dise_table, W1, b1, W2, b2):
    symp_i = symp.astype(jnp.int32)
    label2d = label.astype(jnp.int32).reshape(LROWS, 128)
    h = VPAD // 2
    symp_r = jnp.where(symp_i < h, 2 * symp_i, 2 * (symp_i - h) + 1)
    table_lin = _tc_transpose(symp_table.T).reshape(VPAD, D)
    sum_u, sum_d = _sc_gather_pool(symp_r, label2d, table_lin, dise_table)
    return _mlp(symp_i, sum_u, sum_d, W1, b1, W2, b2)
